# baseline (speedup 1.0000x reference)
import functools

import jax
import jax.numpy as jnp
from jax import lax
from jax.experimental import pallas as pl
from jax.experimental.pallas import tpu as pltpu
from jax.experimental.pallas import tpu_sc as plsc

_NUM_CORES = 2
_NUM_SUBCORES = 16
_NUM_WORKERS = _NUM_CORES * _NUM_SUBCORES
_LANES = 16
_IDX_CHUNK = 128
_SROW = 128
_CH = 128
_NBUF = 4
_GPD = 4


def _project_sc(table, w):
    vocab, dim = table.shape
    vp = 1 << (vocab - 1).bit_length()
    n_rows = vp // _SROW
    chunks_per_tile = n_rows // _NUM_WORKERS
    n_valid_chunks = vocab // _CH

    mesh = plsc.VectorSubcoreMesh(core_axis_name="c", subcore_axis_name="s")

    @functools.partial(
        pl.kernel,
        mesh=mesh,
        out_type=jax.ShapeDtypeStruct((n_rows, _SROW), jnp.float32),
        compiler_params=pltpu.CompilerParams(needs_layout_passes=False),
        scratch_types=[
            pltpu.VMEM((2 * dim,), jnp.float32),
            pltpu.VMEM((chunks_per_tile, _SROW), jnp.float32),
        ] + [pltpu.VMEM((_CH, dim), jnp.float32)] * _NBUF
          + [pltpu.SemaphoreType.DMA] * _NBUF,
    )
    def run(table_hbm, w_hbm, s_hbm, w_v, s_local, *bufs_sems):
        bufs, sems = bufs_sems[:_NBUF], bufs_sems[_NBUF:]
        wid = lax.axis_index("s") * _NUM_CORES + lax.axis_index("c")
        base = wid * chunks_per_tile
        nv = lax.max(0, lax.min(chunks_per_tile, n_valid_chunks - base))
        pltpu.sync_copy(w_hbm, w_v.at[pl.ds(0, dim)])
        pltpu.sync_copy(w_hbm, w_v.at[pl.ds(dim, dim)])

        def issue(c_local, b):
            @pl.when(c_local < nv)
            def _():
                pltpu.async_copy(
                    table_hbm.at[pl.ds((base + c_local) * _CH, _CH), :],
                    bufs[b], sems[b])

        for b in range(_NBUF):
            issue(b, b)

        def ring_body(q, carry):
            for b in range(_NBUF):
                c_local = q * _NBUF + b

                @pl.when(c_local < nv)
                def _(b=b, c_local=c_local):
                    pltpu.make_async_copy(
                        table_hbm.at[pl.ds(0, _CH), :],
                        bufs[b], sems[b]).wait()
                    tb = bufs[b]
                    for gg in range(_CH // (_GPD * _LANES)):
                        dpl = lax.iota(jnp.int32, _LANES)
                        one = jnp.ones((_LANES,), jnp.int32)
                        msk = jnp.full((_LANES,), dim - 1, jnp.int32)
                        rids = [(gg * _GPD + u) * _LANES
                                + lax.iota(jnp.int32, _LANES)
                                for u in range(_GPD)]
                        accs = [jnp.zeros((_LANES,), jnp.float32)
                                for _ in range(_GPD)]
                        for d in range(dim):
                            wd = w_v[pl.ds(d, _LANES)]
                            for u in range(_GPD):
                                v = plsc.load_gather(tb, [rids[u], dpl])
                                accs[u] = accs[u] + v * wd
                            dpl = (dpl + one) & msk
                        for u in range(_GPD):
                            s_local[c_local,
                                    pl.ds((gg * _GPD + u) * _LANES,
                                          _LANES)] = accs[u]
                    issue(c_local + _NBUF, b)
            return carry

        lax.fori_loop(0, chunks_per_tile // _NBUF, ring_body, 0)
        pltpu.sync_copy(
            s_local, s_hbm.at[pl.ds(base, chunks_per_tile)])

    return run(table, w)


def _gather_sc(idx_a, idx_b, s2d):
    n_chunks_total, chunk = idx_a.shape
    batch = n_chunks_total * chunk
    b_per_w = batch // _NUM_WORKERS
    chunks_per_w = b_per_w // chunk
    groups_per_w = b_per_w // _LANES
    groups_per_chunk = chunk // _LANES

    mesh = plsc.VectorSubcoreMesh(core_axis_name="c", subcore_axis_name="s")

    @functools.partial(
        pl.kernel,
        mesh=mesh,
        out_type=jax.ShapeDtypeStruct((batch,), jnp.float32),
        compiler_params=pltpu.CompilerParams(needs_layout_passes=False),
        scratch_types=[
            pltpu.VMEM((chunks_per_w, chunk), jnp.int32),
            pltpu.VMEM((chunks_per_w, chunk), jnp.int32),
            pltpu.VMEM((chunks_per_w, chunk), jnp.int32),
            pltpu.VMEM((b_per_w, _SROW), jnp.float32),
            pltpu.VMEM((b_per_w,), jnp.float32),
            pltpu.VMEM((b_per_w,), jnp.float32),
            pltpu.SemaphoreType.DMA,
        ],
    )
    def run(idx_a_hbm, idx_b_hbm, s_hbm, out_hbm,
            idxa_v, idxb_v, grp_v, rows, acc_v, out_v, sem):
        wid = lax.axis_index("s") * _NUM_CORES + lax.axis_index("c")
        crow = wid * chunks_per_w
        pltpu.sync_copy(idx_a_hbm.at[pl.ds(crow, chunks_per_w)], idxa_v)
        pltpu.sync_copy(idx_b_hbm.at[pl.ds(crow, chunks_per_w)], idxb_v)

        for idx_v in (idxa_v, idxb_v):
            is_b = idx_v is idxb_v

            for r in range(chunks_per_w):
                for c in range(chunk // _LANES):
                    s = pl.ds(c * _LANES, _LANES)
                    grp_v[r, s] = idx_v[r, s] >> 7

            copies = []
            for j in range(chunks_per_w):
                copies.append(pltpu.async_copy(
                    s_hbm.at[grp_v.at[j]],
                    rows.at[pl.ds(j * chunk, chunk)], sem))
            for cpy in copies:
                cpy.wait()

            def group_body(g, carry, idx_v=idx_v, is_b=is_b):
                eid = g * _LANES + lax.iota(jnp.int32, _LANES)
                iv = idx_v[g // groups_per_chunk,
                           pl.ds((g % groups_per_chunk) * _LANES, _LANES)]
                col = iv & (_SROW - 1)
                v = plsc.load_gather(rows, [eid, col])
                s = pl.ds(pl.multiple_of(g * _LANES, _LANES), _LANES)
                if not is_b:
                    acc_v[s] = v
                else:
                    out_v[s] = 1.0 / (1.0 + jnp.exp(v - acc_v[s]))
                return carry

            lax.fori_loop(0, groups_per_w, group_body, 0)

        pltpu.sync_copy(out_v, out_hbm.at[pl.ds(wid * b_per_w, b_per_w)])

    return run(idx_a, idx_b, s2d)


def kernel(item_a, item_b, item_strengths, head_w, head_b):
    batch = item_a.shape[0]
    vocab, dim = item_strengths.shape
    idx_a = item_a.astype(jnp.int32).reshape(batch // _IDX_CHUNK, _IDX_CHUNK)
    idx_b = item_b.astype(jnp.int32).reshape(batch // _IDX_CHUNK, _IDX_CHUNK)
    w = head_w.reshape(dim).astype(jnp.float32)
    s_raw = _project_sc(item_strengths, w).reshape(-1)
    covered = (vocab // _CH) * _CH
    vp = 1 << (vocab - 1).bit_length()
    s_tail = item_strengths[covered:] @ head_w.reshape(dim, 1)
    s_flat = jnp.concatenate(
        [s_raw[:covered], s_tail.reshape(-1),
         jnp.zeros((vp - vocab,), jnp.float32)])
    s2d = s_flat.reshape(vp // _SROW, _SROW)
    out = _gather_sc(idx_a, idx_b, s2d)
    return out.reshape(batch, 1)


# speedup vs baseline: 2.6762x; 2.6762x over previous
import functools

import jax
import jax.numpy as jnp
from jax import lax
from jax.experimental import pallas as pl
from jax.experimental.pallas import tpu as pltpu
from jax.experimental.pallas import tpu_sc as plsc

_NUM_CORES = 2
_NUM_SUBCORES = 16
_NUM_WORKERS = _NUM_CORES * _NUM_SUBCORES
_LANES = 16
_IDX_CHUNK = 128
_NSEM = 4


def _bt_sc_call(idx_a, idx_b, table, w):
    n_chunks_total, chunk = idx_a.shape
    batch = n_chunks_total * chunk
    dim = table.shape[1]
    b_per_w = batch // _NUM_WORKERS
    chunks_per_w = b_per_w // chunk
    groups_per_w = b_per_w // _LANES
    groups_per_chunk = chunk // _LANES

    mesh = plsc.VectorSubcoreMesh(core_axis_name="c", subcore_axis_name="s")

    @functools.partial(
        pl.kernel,
        mesh=mesh,
        out_type=jax.ShapeDtypeStruct((batch,), jnp.float32),
        compiler_params=pltpu.CompilerParams(needs_layout_passes=False),
        scratch_types=[
            pltpu.VMEM((chunks_per_w, chunk), jnp.int32),
            pltpu.VMEM((chunks_per_w, chunk), jnp.int32),
            pltpu.VMEM((2 * dim,), jnp.float32),
            pltpu.VMEM((b_per_w, dim), jnp.float32),
            pltpu.VMEM((b_per_w,), jnp.float32),
            pltpu.VMEM((b_per_w,), jnp.float32),
        ] + [pltpu.SemaphoreType.DMA] * _NSEM,
    )
    def run(idx_a_hbm, idx_b_hbm, table_hbm, w_hbm, out_hbm,
            idxa_v, idxb_v, w_v, rows, acc_v, out_v, *sems):
        wid = lax.axis_index("s") * _NUM_CORES + lax.axis_index("c")
        crow = wid * chunks_per_w
        pltpu.sync_copy(idx_a_hbm.at[pl.ds(crow, chunks_per_w)], idxa_v)
        pltpu.sync_copy(idx_b_hbm.at[pl.ds(crow, chunks_per_w)], idxb_v)
        pltpu.sync_copy(w_hbm, w_v.at[pl.ds(0, dim)])
        pltpu.sync_copy(w_hbm, w_v.at[pl.ds(dim, dim)])

        for idx_v in (idxa_v, idxb_v):
            is_b = idx_v is idxb_v

            def issue_body(q, carry, idx_v=idx_v):
                ivs = []
                for n in range(_NSEM):
                    g = q * _NSEM + n
                    ivs.append(idx_v[g // groups_per_chunk,
                                     pl.ds((g % groups_per_chunk) * _LANES,
                                           _LANES)])
                for l in range(_LANES):
                    for n in range(_NSEM):
                        g = q * _NSEM + n
                        pltpu.async_copy(
                            table_hbm.at[pl.ds(ivs[n][l], 1), :],
                            rows.at[pl.ds(g * _LANES + l, 1), :], sems[n])
                return carry

            lax.fori_loop(0, groups_per_w // _NSEM, issue_body, 0)
            part = b_per_w // _NSEM
            for n in range(_NSEM):
                pltpu.make_async_copy(
                    table_hbm.at[pl.ds(0, part), :],
                    rows.at[pl.ds(n * part, part), :], sems[n]).wait()

            def group_body(g, carry, is_b=is_b, idx_v=idx_v):
                rid = g * _LANES + lax.iota(jnp.int32, _LANES)
                dpl = lax.iota(jnp.int32, _LANES)
                one = jnp.ones((_LANES,), jnp.int32)
                msk = jnp.full((_LANES,), dim - 1, jnp.int32)
                acc = jnp.zeros((_LANES,), jnp.float32)
                for d in range(dim):
                    v = plsc.load_gather(rows, [rid, dpl])
                    wd = w_v[pl.ds(d, _LANES)]
                    acc = acc + v * wd
                    dpl = (dpl + one) & msk
                s = pl.ds(pl.multiple_of(g * _LANES, _LANES), _LANES)
                if not is_b:
                    acc_v[s] = acc
                else:
                    out_v[s] = 1.0 / (1.0 + jnp.exp(acc - acc_v[s]))
                return carry

            lax.fori_loop(0, groups_per_w, group_body, 0)

        pltpu.sync_copy(out_v, out_hbm.at[pl.ds(wid * b_per_w, b_per_w)])

    return run(idx_a, idx_b, table, w)


def kernel(item_a, item_b, item_strengths, head_w, head_b):
    batch = item_a.shape[0]
    dim = item_strengths.shape[1]
    idx_a = item_a.astype(jnp.int32).reshape(batch // _IDX_CHUNK, _IDX_CHUNK)
    idx_b = item_b.astype(jnp.int32).reshape(batch // _IDX_CHUNK, _IDX_CHUNK)
    w = head_w.reshape(dim).astype(jnp.float32)
    out = _bt_sc_call(idx_a, idx_b, item_strengths, w)
    return out.reshape(batch, 1)
